# baseline (speedup 1.0000x reference)
import jax
import jax.numpy as jnp
from jax.experimental import pallas as pl
from jax.experimental.pallas import tpu as pltpu

_NUM_CLASSES = 80
_NUM_ANCHORS = 5
_AW = (1.3221, 3.19275, 5.05587, 9.47112, 11.2364)
_AH = (1.73145, 4.00944, 8.09892, 4.84053, 10.0071)
_OBJECT_SCALE = 5.0
_MAX_GT = 50
_NB, _NH, _NW = 16, 64, 64
_NHS, _NL = 32, 128
_TPAD = 128
_LOG80 = 4.382026634673881

(_R_GXL, _R_GXR, _R_GYT, _R_GYB, _R_G375, _R_JJ, _R_BNV, _R_II,
 _R_TX, _R_TY, _R_TW, _R_TH, _R_TCONF, _R_TCLS, _R_BN) = range(15)
_N_ROWS = 15


def _anchor_iou(gw, gh, aw, ah):
    mx = jnp.minimum(-0.5 * gw, -0.5 * aw)
    Mx = jnp.maximum(0.5 * gw, 0.5 * aw)
    my = jnp.minimum(-0.5 * gh, -0.5 * ah)
    My = jnp.maximum(0.5 * gh, 0.5 * ah)
    cw = gw + aw - (Mx - mx)
    ch = gh + ah - (My - my)
    inter = jnp.where((cw > 0) & (ch > 0), cw * ch, 0.0)
    union = gw * gh + aw * ah - inter
    return inter / union


def _sel_anchor(v, table):
    r = table[-1]
    for k in range(_NUM_ANCHORS - 2, -1, -1):
        r = jnp.where(v == k, table[k], r)
    return r


def _region_loss_kernel(tgt_s, tgt_v, raw_blk, cls_hbm, part,
                        sc, swin, wrow, gbuf, sem_g, sem_w):
    b = pl.program_id(0)
    f32 = jnp.float32

    def body(t, valid_c):
        t0 = tgt_s[b, 0 * _TPAD + t]
        t1 = tgt_s[b, 1 * _TPAD + t]
        t2 = tgt_s[b, 2 * _TPAD + t]
        t3 = tgt_s[b, 3 * _TPAD + t]
        t4 = tgt_s[b, 4 * _TPAD + t]
        valid = valid_c * jnp.where(t1 != 0.0, 1.0, 0.0)
        gx = t1 * _NW
        gy = t2 * _NH
        gw = t3 * _NW
        gh = t4 * _NH
        best = _anchor_iou(gw, gh, _AW[0], _AH[0])
        bn = f32(0.0)
        for k in range(1, _NUM_ANCHORS):
            iou_k = _anchor_iou(gw, gh, _AW[k], _AH[k])
            upd = iou_k > best
            best = jnp.where(upd, iou_k, best)
            bn = jnp.where(upd, f32(k), bn)
        ii = jnp.floor(gx)
        jj = jnp.floor(gy)
        awb = _sel_anchor(bn, _AW)
        ahb = _sel_anchor(bn, _AH)
        sc[_R_GXL, t] = gx - 0.5 * gw
        sc[_R_GXR, t] = gx + 0.5 * gw
        sc[_R_GYT, t] = gy - 0.5 * gh
        sc[_R_GYB, t] = gy + 0.5 * gh
        sc[_R_G375, t] = jnp.where(valid > 0, 0.375 * gw * gh, f32(jnp.inf))
        sc[_R_JJ, t] = jj
        sc[_R_BNV, t] = jnp.where(valid > 0, bn, f32(-1.0))
        sc[_R_II, t] = ii
        sc[_R_TX, t] = gx - ii
        sc[_R_TY, t] = gy - jj
        sc[_R_TW, t] = jnp.log(gw / awb)
        sc[_R_TH, t] = jnp.log(gh / ahb)
        sc[_R_TCONF, t] = best
        sc[_R_TCLS, t] = jnp.floor(t0)
        sc[_R_BN, t] = bn
        return valid

    jax.lax.fori_loop(0, _MAX_GT, body, f32(1.0))

    for t in range(_MAX_GT):
        c0 = sc[_R_BN, t].astype(jnp.int32) * (5 + _NUM_CLASSES)
        jj_i = sc[_R_JJ, t].astype(jnp.int32)
        pltpu.make_async_copy(
            cls_hbm.at[b, pl.ds(c0, 5 + _NUM_CLASSES), jj_i],
            gbuf.at[t], sem_g.at[t]).start()

    iota_t = jax.lax.broadcasted_iota(jnp.int32, (1, _TPAD), 1).astype(f32)
    t1v = tgt_v[0, 1:2, :]
    t2v = tgt_v[0, 2:3, :]
    t3v = tgt_v[0, 3:4, :]
    t4v = tgt_v[0, 4:5, :]
    fz = jnp.min(jnp.where(t1v == 0.0, iota_t, f32(_TPAD)), axis=1,
                 keepdims=True)
    validv = jnp.where(iota_t < fz, 1.0, 0.0)
    gwv = t3v * _NW
    ghv = t4v * _NH
    bestv = _anchor_iou(gwv, ghv, _AW[0], _AH[0])
    bnv = jnp.zeros((1, _TPAD), f32)
    for k in range(1, _NUM_ANCHORS):
        iou_k = _anchor_iou(gwv, ghv, _AW[k], _AH[k])
        updv = iou_k > bestv
        bestv = jnp.where(updv, iou_k, bestv)
        bnv = jnp.where(updv, f32(k), bnv)
    iiv = jnp.floor(t1v * _NW)
    jjv = jnp.floor(t2v * _NH)
    keyv = jnp.where(validv > 0, (bnv * 64 + jjv) * 64 + iiv, f32(-1.0))
    ones_r = jnp.ones((1, _TPAD), f32)
    kT = jax.lax.dot_general(keyv, ones_r, (((0,), (0,)), ((), ())),
                             preferred_element_type=f32)
    kB = jnp.broadcast_to(keyv, (_TPAD, _TPAD))
    ridx = jax.lax.broadcasted_iota(jnp.int32, (_TPAD, _TPAD), 0)
    cidx = jax.lax.broadcasted_iota(jnp.int32, (_TPAD, _TPAD), 1)
    loser = jnp.max(jnp.where((kT == kB) & (ridx > cidx), 1.0, 0.0),
                    axis=0, keepdims=True)
    wrow[...] = validv * (1.0 - loser)
    pltpu.make_async_copy(wrow, swin, sem_w).start()

    li = jax.lax.broadcasted_iota(jnp.int32, (_NHS, _NL), 1)
    si = jax.lax.broadcasted_iota(jnp.int32, (_NHS, _NL), 0)
    coli = (li & 63).astype(f32)
    rowi = (si * 2 + (li >> 6)).astype(f32)
    zeros = jnp.zeros((_NHS, _NL), f32)

    lconf = jnp.zeros((1, _NL), f32)
    for a in range(_NUM_ANCHORS):
        af = f32(a)
        sx = jax.nn.sigmoid(raw_blk[0, a, 0])
        sy = jax.nn.sigmoid(raw_blk[0, a, 1])
        pw = jnp.exp(raw_blk[0, a, 2]) * _AW[a]
        ph = jnp.exp(raw_blk[0, a, 3]) * _AH[a]
        px = sx + coli
        py = sy + rowi
        pxl = px - 0.5 * pw
        pxr = px + 0.5 * pw
        pyt = py - 0.5 * ph
        pyb = py + 0.5 * ph
        parea375 = (0.375 * pw) * ph
        sil = zeros > 1.0
        maskc = zeros
        tcfc = zeros
        for t in range(_MAX_GT):
            ox = jnp.minimum(pxr, sc[_R_GXR, t]) - jnp.maximum(pxl, sc[_R_GXL, t])
            oy = jnp.minimum(pyb, sc[_R_GYB, t]) - jnp.maximum(pyt, sc[_R_GYT, t])
            inter = jnp.maximum(ox, 0.0) * oy
            sil = sil | (inter > parea375 + sc[_R_G375, t])
            ii_m = jnp.where(sc[_R_BNV, t] == af, sc[_R_II, t], f32(-1.0))
            hit = (rowi == sc[_R_JJ, t]) & (coli == ii_m)
            maskc = jnp.where(hit, 1.0, maskc)
            tcfc = jnp.where(hit, sc[_R_TCONF, t], tcfc)
        conf = jax.nn.sigmoid(raw_blk[0, a, 4])
        scale = jnp.where(maskc > 0, _OBJECT_SCALE, jnp.where(sil, 0.0, 1.0))
        dconf = scale * (conf - tcfc)
        lconf = lconf + jnp.sum(dconf * dconf, axis=0, keepdims=True)

    pltpu.make_async_copy(wrow, swin, sem_w).wait()

    lane64 = jax.lax.broadcasted_iota(jnp.int32, (1, _NW), 1).astype(f32)
    si80 = jax.lax.broadcasted_iota(jnp.int32, (_NUM_CLASSES, _NW), 0).astype(f32)
    lx = jnp.zeros((1, _NW), f32)
    ly = jnp.zeros((1, _NW), f32)
    lw = jnp.zeros((1, _NW), f32)
    lh = jnp.zeros((1, _NW), f32)
    lcls = jnp.zeros((1, _NW), f32)
    for t in range(_MAX_GT):
        pltpu.make_async_copy(gbuf.at[t], gbuf.at[t], sem_g.at[t]).wait()
        wv = swin[0, t]
        ii_g = jnp.where(wv > 0, sc[_R_II, t], f32(-1.0))
        lm = lane64 == ii_g
        sxr = jax.nn.sigmoid(gbuf[t, 0:1, :])
        syr = jax.nn.sigmoid(gbuf[t, 1:2, :])
        rwr = gbuf[t, 2:3, :]
        rhr = gbuf[t, 3:4, :]
        lx = lx + jnp.where(lm, (sxr - sc[_R_TX, t]) ** 2, 0.0)
        ly = ly + jnp.where(lm, (syr - sc[_R_TY, t]) ** 2, 0.0)
        lw = lw + jnp.where(lm, (rwr - sc[_R_TW, t]) ** 2, 0.0)
        lh = lh + jnp.where(lm, (rhr - sc[_R_TH, t]) ** 2, 0.0)
        gcls = gbuf[t, 5:5 + _NUM_CLASSES, :]
        lse = jnp.log(jnp.sum(jnp.exp(gcls), axis=0, keepdims=True))
        pick = jnp.sum(jnp.where(si80 == sc[_R_TCLS, t], gcls, 0.0),
                       axis=0, keepdims=True)
        lcls = lcls + jnp.where(lm, (lse - pick) - _LOG80, 0.0)

    def pad128(r):
        return jnp.concatenate([r, jnp.zeros((1, _NL - _NW), f32)], axis=1)

    ccls = pad128(lcls) + f32(_NUM_ANCHORS * _NH * _NW * _LOG80 / _NL)

    stacked = jnp.concatenate(
        [pad128(lx), pad128(ly), pad128(lw), pad128(lh), lconf, ccls,
         validv, jnp.zeros((1, _NL), f32)], axis=0)
    part[0] = stacked


def kernel(output, target):
    nB = output.shape[0]
    out5 = output.reshape(nB, _NUM_ANCHORS, 5 + _NUM_CLASSES, _NH, _NW)
    raw = out5[:, :, :5].reshape(nB, _NUM_ANCHORS, 5, _NHS, _NL)
    tgt = target.reshape(nB, _MAX_GT, 5).transpose(0, 2, 1)
    tgt = jnp.pad(tgt, ((0, 0), (0, 0), (0, _TPAD - _MAX_GT)))
    tgt_flat = tgt.reshape(nB, 5 * _TPAD)

    part = pl.pallas_call(
        _region_loss_kernel,
        out_shape=jax.ShapeDtypeStruct((nB, 8, _NL), jnp.float32),
        grid=(nB,),
        in_specs=[
            pl.BlockSpec(memory_space=pltpu.SMEM),
            pl.BlockSpec((1, 5, _TPAD), lambda b: (b, 0, 0)),
            pl.BlockSpec((1, _NUM_ANCHORS, 5, _NHS, _NL), lambda b: (b, 0, 0, 0, 0)),
            pl.BlockSpec(memory_space=pl.ANY),
        ],
        out_specs=pl.BlockSpec((1, 8, _NL), lambda b: (b, 0, 0)),
        scratch_shapes=[
            pltpu.SMEM((_N_ROWS, _TPAD), jnp.float32),
            pltpu.SMEM((1, _TPAD), jnp.float32),
            pltpu.VMEM((1, _TPAD), jnp.float32),
            pltpu.VMEM((_MAX_GT, 5 + _NUM_CLASSES, _NW), jnp.float32),
            pltpu.SemaphoreType.DMA((_MAX_GT,)),
            pltpu.SemaphoreType.DMA,
        ],
        compiler_params=pltpu.CompilerParams(
            dimension_semantics=("parallel",),
        ),
        name="region_loss_sparse",
    )(tgt_flat, tgt, raw, output)

    sums = jnp.sum(part, axis=(0, 2))
    ngt = sums[6]
    return (sums[0] + sums[1] + sums[2] + sums[3] + sums[4] + sums[5]) / ngt


# speedup vs baseline: 1.9433x; 1.9433x over previous
import jax
import jax.numpy as jnp
from jax.experimental import pallas as pl
from jax.experimental.pallas import tpu as pltpu

_NUM_CLASSES = 80
_NUM_ANCHORS = 5
_AW = (1.3221, 3.19275, 5.05587, 9.47112, 11.2364)
_AH = (1.73145, 4.00944, 8.09892, 4.84053, 10.0071)
_OBJECT_SCALE = 5.0
_MAX_GT = 50
_NB, _NH, _NW = 16, 64, 64
_NHS, _NL = 32, 128
_TPAD = 128
_LOG80 = 4.382026634673881

(_R_GXL, _R_GXR, _R_GYT, _R_GYB, _R_G375, _R_JJ, _R_BNV, _R_II,
 _R_TX, _R_TY, _R_TW, _R_TH, _R_TCONF, _R_TCLS, _R_BN) = range(15)
_N_ROWS = 15


def _anchor_iou(gw, gh, aw, ah):
    mx = jnp.minimum(-0.5 * gw, -0.5 * aw)
    Mx = jnp.maximum(0.5 * gw, 0.5 * aw)
    my = jnp.minimum(-0.5 * gh, -0.5 * ah)
    My = jnp.maximum(0.5 * gh, 0.5 * ah)
    cw = gw + aw - (Mx - mx)
    ch = gh + ah - (My - my)
    inter = jnp.where((cw > 0) & (ch > 0), cw * ch, 0.0)
    union = gw * gh + aw * ah - inter
    return inter / union


def _sel_anchor(v, table):
    r = table[-1]
    for k in range(_NUM_ANCHORS - 2, -1, -1):
        r = jnp.where(v == k, table[k], r)
    return r


def _region_loss_kernel(tgt_s, tgt_v, raw_blk, cls_hbm, part,
                        sc, swin, wrow, gbuf, sem_g, sem_w):
    b = pl.program_id(0)
    f32 = jnp.float32

    def body(t, valid_c):
        t0 = tgt_s[b, 0 * _TPAD + t]
        t1 = tgt_s[b, 1 * _TPAD + t]
        t2 = tgt_s[b, 2 * _TPAD + t]
        t3 = tgt_s[b, 3 * _TPAD + t]
        t4 = tgt_s[b, 4 * _TPAD + t]
        valid = valid_c * jnp.where(t1 != 0.0, 1.0, 0.0)
        gx = t1 * _NW
        gy = t2 * _NH
        gw = t3 * _NW
        gh = t4 * _NH
        best = _anchor_iou(gw, gh, _AW[0], _AH[0])
        bn = f32(0.0)
        for k in range(1, _NUM_ANCHORS):
            iou_k = _anchor_iou(gw, gh, _AW[k], _AH[k])
            upd = iou_k > best
            best = jnp.where(upd, iou_k, best)
            bn = jnp.where(upd, f32(k), bn)
        ii = jnp.floor(gx)
        jj = jnp.floor(gy)
        awb = _sel_anchor(bn, _AW)
        ahb = _sel_anchor(bn, _AH)
        sc[_R_GXL, t] = gx - 0.5 * gw
        sc[_R_GXR, t] = gx + 0.5 * gw
        sc[_R_GYT, t] = gy - 0.5 * gh
        sc[_R_GYB, t] = gy + 0.5 * gh
        sc[_R_G375, t] = jnp.where(valid > 0, 0.375 * gw * gh, f32(jnp.inf))
        sc[_R_JJ, t] = jj
        sc[_R_BNV, t] = jnp.where(valid > 0, bn, f32(-1.0))
        sc[_R_II, t] = ii
        sc[_R_TX, t] = gx - ii
        sc[_R_TY, t] = gy - jj
        sc[_R_TW, t] = jnp.log(gw / awb)
        sc[_R_TH, t] = jnp.log(gh / ahb)
        sc[_R_TCONF, t] = best
        sc[_R_TCLS, t] = jnp.floor(t0)
        sc[_R_BN, t] = bn
        return valid

    jax.lax.fori_loop(0, _MAX_GT, body, f32(1.0))

    for t in range(_MAX_GT):
        bn_i = sc[_R_BN, t].astype(jnp.int32)
        jj_i = sc[_R_JJ, t].astype(jnp.int32)
        pltpu.make_async_copy(
            cls_hbm.at[b, bn_i, pl.ds(0, 5 + _NUM_CLASSES), jj_i],
            gbuf.at[t], sem_g.at[t]).start()

    iota_t = jax.lax.broadcasted_iota(jnp.int32, (1, _TPAD), 1).astype(f32)
    t1v = tgt_v[0, 1:2, :]
    t2v = tgt_v[0, 2:3, :]
    t3v = tgt_v[0, 3:4, :]
    t4v = tgt_v[0, 4:5, :]
    fz = jnp.min(jnp.where(t1v == 0.0, iota_t, f32(_TPAD)), axis=1,
                 keepdims=True)
    validv = jnp.where(iota_t < fz, 1.0, 0.0)
    gwv = t3v * _NW
    ghv = t4v * _NH
    bestv = _anchor_iou(gwv, ghv, _AW[0], _AH[0])
    bnv = jnp.zeros((1, _TPAD), f32)
    for k in range(1, _NUM_ANCHORS):
        iou_k = _anchor_iou(gwv, ghv, _AW[k], _AH[k])
        updv = iou_k > bestv
        bestv = jnp.where(updv, iou_k, bestv)
        bnv = jnp.where(updv, f32(k), bnv)
    iiv = jnp.floor(t1v * _NW)
    jjv = jnp.floor(t2v * _NH)
    keyv = jnp.where(validv > 0, (bnv * 64 + jjv) * 64 + iiv, f32(-1.0))
    ones_r = jnp.ones((1, _TPAD), f32)
    kT = jax.lax.dot_general(keyv, ones_r, (((0,), (0,)), ((), ())),
                             preferred_element_type=f32)
    kB = jnp.broadcast_to(keyv, (_TPAD, _TPAD))
    ridx = jax.lax.broadcasted_iota(jnp.int32, (_TPAD, _TPAD), 0)
    cidx = jax.lax.broadcasted_iota(jnp.int32, (_TPAD, _TPAD), 1)
    loser = jnp.max(jnp.where((kT == kB) & (ridx > cidx), 1.0, 0.0),
                    axis=0, keepdims=True)
    wrow[...] = validv * (1.0 - loser)
    pltpu.make_async_copy(wrow, swin, sem_w).start()

    li = jax.lax.broadcasted_iota(jnp.int32, (_NHS, _NL), 1)
    si = jax.lax.broadcasted_iota(jnp.int32, (_NHS, _NL), 0)
    coli = (li & 63).astype(f32)
    rowi = (si * 2 + (li >> 6)).astype(f32)
    zeros = jnp.zeros((_NHS, _NL), f32)

    lconf = jnp.zeros((1, _NL), f32)
    for a in range(_NUM_ANCHORS):
        af = f32(a)
        sx = jax.nn.sigmoid(raw_blk[0, a, 0])
        sy = jax.nn.sigmoid(raw_blk[0, a, 1])
        pw = jnp.exp(raw_blk[0, a, 2]) * _AW[a]
        ph = jnp.exp(raw_blk[0, a, 3]) * _AH[a]
        px = sx + coli
        py = sy + rowi
        pxl = px - 0.5 * pw
        pxr = px + 0.5 * pw
        pyt = py - 0.5 * ph
        pyb = py + 0.5 * ph
        parea375 = (0.375 * pw) * ph
        sil = zeros > 1.0
        maskc = zeros
        tcfc = zeros
        for t in range(_MAX_GT):
            ox = jnp.minimum(pxr, sc[_R_GXR, t]) - jnp.maximum(pxl, sc[_R_GXL, t])
            oy = jnp.minimum(pyb, sc[_R_GYB, t]) - jnp.maximum(pyt, sc[_R_GYT, t])
            inter = jnp.maximum(ox, 0.0) * oy
            sil = sil | (inter > parea375 + sc[_R_G375, t])
            ii_m = jnp.where(sc[_R_BNV, t] == af, sc[_R_II, t], f32(-1.0))
            hit = (rowi == sc[_R_JJ, t]) & (coli == ii_m)
            maskc = jnp.where(hit, 1.0, maskc)
            tcfc = jnp.where(hit, sc[_R_TCONF, t], tcfc)
        conf = jax.nn.sigmoid(raw_blk[0, a, 4])
        scale = jnp.where(maskc > 0, _OBJECT_SCALE, jnp.where(sil, 0.0, 1.0))
        dconf = scale * (conf - tcfc)
        lconf = lconf + jnp.sum(dconf * dconf, axis=0, keepdims=True)

    pltpu.make_async_copy(wrow, swin, sem_w).wait()

    lane64 = jax.lax.broadcasted_iota(jnp.int32, (1, _NW), 1).astype(f32)
    si80 = jax.lax.broadcasted_iota(jnp.int32, (_NUM_CLASSES, _NW), 0).astype(f32)
    lx = jnp.zeros((1, _NW), f32)
    ly = jnp.zeros((1, _NW), f32)
    lw = jnp.zeros((1, _NW), f32)
    lh = jnp.zeros((1, _NW), f32)
    lcls = jnp.zeros((1, _NW), f32)
    for t in range(_MAX_GT):
        pltpu.make_async_copy(gbuf.at[t], gbuf.at[t], sem_g.at[t]).wait()
        wv = swin[0, t]
        ii_g = jnp.where(wv > 0, sc[_R_II, t], f32(-1.0))
        lm = lane64 == ii_g
        sxr = jax.nn.sigmoid(gbuf[t, 0:1, :])
        syr = jax.nn.sigmoid(gbuf[t, 1:2, :])
        rwr = gbuf[t, 2:3, :]
        rhr = gbuf[t, 3:4, :]
        lx = lx + jnp.where(lm, (sxr - sc[_R_TX, t]) ** 2, 0.0)
        ly = ly + jnp.where(lm, (syr - sc[_R_TY, t]) ** 2, 0.0)
        lw = lw + jnp.where(lm, (rwr - sc[_R_TW, t]) ** 2, 0.0)
        lh = lh + jnp.where(lm, (rhr - sc[_R_TH, t]) ** 2, 0.0)
        gcls = gbuf[t, 5:5 + _NUM_CLASSES, :]
        lse = jnp.log(jnp.sum(jnp.exp(gcls), axis=0, keepdims=True))
        pick = jnp.sum(jnp.where(si80 == sc[_R_TCLS, t], gcls, 0.0),
                       axis=0, keepdims=True)
        lcls = lcls + jnp.where(lm, (lse - pick) - _LOG80, 0.0)

    def pad128(r):
        return jnp.concatenate([r, jnp.zeros((1, _NL - _NW), f32)], axis=1)

    ccls = pad128(lcls) + f32(_NUM_ANCHORS * _NH * _NW * _LOG80 / _NL)

    stacked = jnp.concatenate(
        [pad128(lx), pad128(ly), pad128(lw), pad128(lh), lconf, ccls,
         validv, jnp.zeros((1, _NL), f32)], axis=0)
    part[0] = stacked


def kernel(output, target):
    nB = output.shape[0]
    out5 = output.reshape(nB, _NUM_ANCHORS, 5 + _NUM_CLASSES, _NH, _NW)
    raw = out5[:, :, :5].reshape(nB, _NUM_ANCHORS, 5, _NHS, _NL)
    tgt = target.reshape(nB, _MAX_GT, 5).transpose(0, 2, 1)
    tgt = jnp.pad(tgt, ((0, 0), (0, 0), (0, _TPAD - _MAX_GT)))
    tgt_flat = tgt.reshape(nB, 5 * _TPAD)

    part = pl.pallas_call(
        _region_loss_kernel,
        out_shape=jax.ShapeDtypeStruct((nB, 8, _NL), jnp.float32),
        grid=(nB,),
        in_specs=[
            pl.BlockSpec(memory_space=pltpu.SMEM),
            pl.BlockSpec((1, 5, _TPAD), lambda b: (b, 0, 0)),
            pl.BlockSpec((1, _NUM_ANCHORS, 5, _NHS, _NL), lambda b: (b, 0, 0, 0, 0)),
            pl.BlockSpec(memory_space=pl.ANY),
        ],
        out_specs=pl.BlockSpec((1, 8, _NL), lambda b: (b, 0, 0)),
        scratch_shapes=[
            pltpu.SMEM((_N_ROWS, _TPAD), jnp.float32),
            pltpu.SMEM((1, _TPAD), jnp.float32),
            pltpu.VMEM((1, _TPAD), jnp.float32),
            pltpu.VMEM((_MAX_GT, 5 + _NUM_CLASSES, _NW), jnp.float32),
            pltpu.SemaphoreType.DMA((_MAX_GT,)),
            pltpu.SemaphoreType.DMA,
        ],
        compiler_params=pltpu.CompilerParams(
            dimension_semantics=("parallel",),
        ),
        name="region_loss_sparse",
    )(tgt_flat, tgt, raw, out5)

    sums = jnp.sum(part, axis=(0, 2))
    ngt = sums[6]
    return (sums[0] + sums[1] + sums[2] + sums[3] + sums[4] + sums[5]) / ngt


# speedup vs baseline: 2.0302x; 1.0448x over previous
import jax
import jax.numpy as jnp
from jax.experimental import pallas as pl
from jax.experimental.pallas import tpu as pltpu

_NUM_CLASSES = 80
_NUM_ANCHORS = 5
_AW = (1.3221, 3.19275, 5.05587, 9.47112, 11.2364)
_AH = (1.73145, 4.00944, 8.09892, 4.84053, 10.0071)
_OBJECT_SCALE = 5.0
_MAX_GT = 50
_NB, _NH, _NW = 16, 64, 64
_NHS, _NL = 32, 128
_TPAD = 128
_LOG80 = 4.382026634673881

(_R_GXL, _R_GXR, _R_GYT, _R_GYB, _R_G375, _R_JJ, _R_BNV, _R_II,
 _R_TX, _R_TY, _R_TW, _R_TH, _R_TCONF, _R_TCLS, _R_BN) = range(15)
_N_ROWS = 15


def _anchor_iou(gw, gh, aw, ah):
    mx = jnp.minimum(-0.5 * gw, -0.5 * aw)
    Mx = jnp.maximum(0.5 * gw, 0.5 * aw)
    my = jnp.minimum(-0.5 * gh, -0.5 * ah)
    My = jnp.maximum(0.5 * gh, 0.5 * ah)
    cw = gw + aw - (Mx - mx)
    ch = gh + ah - (My - my)
    inter = jnp.where((cw > 0) & (ch > 0), cw * ch, 0.0)
    union = gw * gh + aw * ah - inter
    return inter / union


def _sel_anchor(v, table):
    r = table[-1]
    for k in range(_NUM_ANCHORS - 2, -1, -1):
        r = jnp.where(v == k, table[k], r)
    return r


def _region_loss_kernel(tgt_s, tgt_v, raw_blk, cls_hbm, part,
                        sc, swin, wrow, gbuf, sem_g, sem_w):
    b = pl.program_id(0)
    f32 = jnp.float32

    def body(t, valid_c):
        t0 = tgt_s[b, 0 * _TPAD + t]
        t1 = tgt_s[b, 1 * _TPAD + t]
        t2 = tgt_s[b, 2 * _TPAD + t]
        t3 = tgt_s[b, 3 * _TPAD + t]
        t4 = tgt_s[b, 4 * _TPAD + t]
        valid = valid_c * jnp.where(t1 != 0.0, 1.0, 0.0)
        gx = t1 * _NW
        gy = t2 * _NH
        gw = t3 * _NW
        gh = t4 * _NH
        best = _anchor_iou(gw, gh, _AW[0], _AH[0])
        bn = f32(0.0)
        for k in range(1, _NUM_ANCHORS):
            iou_k = _anchor_iou(gw, gh, _AW[k], _AH[k])
            upd = iou_k > best
            best = jnp.where(upd, iou_k, best)
            bn = jnp.where(upd, f32(k), bn)
        ii = jnp.floor(gx)
        jj = jnp.floor(gy)
        awb = _sel_anchor(bn, _AW)
        ahb = _sel_anchor(bn, _AH)
        sc[_R_GXL, t] = gx - 0.5 * gw
        sc[_R_GXR, t] = gx + 0.5 * gw
        sc[_R_GYT, t] = gy - 0.5 * gh
        sc[_R_GYB, t] = gy + 0.5 * gh
        sc[_R_G375, t] = jnp.where(valid > 0, 0.375 * gw * gh, f32(jnp.inf))
        sc[_R_JJ, t] = jj
        sc[_R_BNV, t] = jnp.where(valid > 0, bn, f32(-1.0))
        sc[_R_II, t] = ii
        sc[_R_TX, t] = gx - ii
        sc[_R_TY, t] = gy - jj
        sc[_R_TW, t] = jnp.log(gw / awb)
        sc[_R_TH, t] = jnp.log(gh / ahb)
        sc[_R_TCONF, t] = best
        sc[_R_TCLS, t] = jnp.floor(t0)
        sc[_R_BN, t] = bn
        return valid

    jax.lax.fori_loop(0, _MAX_GT, body, f32(1.0))

    for t in range(_MAX_GT):
        bn_i = sc[_R_BN, t].astype(jnp.int32)
        jj_i = sc[_R_JJ, t].astype(jnp.int32)
        pltpu.make_async_copy(
            cls_hbm.at[b, bn_i, pl.ds(0, 5 + _NUM_CLASSES), jj_i],
            gbuf.at[t], sem_g).start()

    iota_t = jax.lax.broadcasted_iota(jnp.int32, (1, _TPAD), 1).astype(f32)
    t1v = tgt_v[0, 1:2, :]
    t2v = tgt_v[0, 2:3, :]
    t3v = tgt_v[0, 3:4, :]
    t4v = tgt_v[0, 4:5, :]
    fz = jnp.min(jnp.where(t1v == 0.0, iota_t, f32(_TPAD)), axis=1,
                 keepdims=True)
    validv = jnp.where(iota_t < fz, 1.0, 0.0)
    gwv = t3v * _NW
    ghv = t4v * _NH
    bestv = _anchor_iou(gwv, ghv, _AW[0], _AH[0])
    bnv = jnp.zeros((1, _TPAD), f32)
    for k in range(1, _NUM_ANCHORS):
        iou_k = _anchor_iou(gwv, ghv, _AW[k], _AH[k])
        updv = iou_k > bestv
        bestv = jnp.where(updv, iou_k, bestv)
        bnv = jnp.where(updv, f32(k), bnv)
    iiv = jnp.floor(t1v * _NW)
    jjv = jnp.floor(t2v * _NH)
    keyv = jnp.where(validv > 0, (bnv * 64 + jjv) * 64 + iiv, f32(-1.0))
    ones_r = jnp.ones((1, _TPAD), f32)
    kT = jax.lax.dot_general(keyv, ones_r, (((0,), (0,)), ((), ())),
                             preferred_element_type=f32)
    kB = jnp.broadcast_to(keyv, (_TPAD, _TPAD))
    ridx = jax.lax.broadcasted_iota(jnp.int32, (_TPAD, _TPAD), 0)
    cidx = jax.lax.broadcasted_iota(jnp.int32, (_TPAD, _TPAD), 1)
    loser = jnp.max(jnp.where((kT == kB) & (ridx > cidx), 1.0, 0.0),
                    axis=0, keepdims=True)
    wrow[...] = validv * (1.0 - loser)
    pltpu.make_async_copy(wrow, swin, sem_w).start()

    li = jax.lax.broadcasted_iota(jnp.int32, (_NHS, _NL), 1)
    si = jax.lax.broadcasted_iota(jnp.int32, (_NHS, _NL), 0)
    coli = (li & 63).astype(f32)
    rowi = (si * 2 + (li >> 6)).astype(f32)
    zeros = jnp.zeros((_NHS, _NL), f32)

    lconf = jnp.zeros((1, _NL), f32)
    for a in range(_NUM_ANCHORS):
        af = f32(a)
        sx = jax.nn.sigmoid(raw_blk[0, a, 0])
        sy = jax.nn.sigmoid(raw_blk[0, a, 1])
        pw = jnp.exp(raw_blk[0, a, 2]) * _AW[a]
        ph = jnp.exp(raw_blk[0, a, 3]) * _AH[a]
        px = sx + coli
        py = sy + rowi
        pxl = px - 0.5 * pw
        pxr = px + 0.5 * pw
        pyt = py - 0.5 * ph
        pyb = py + 0.5 * ph
        parea375 = (0.375 * pw) * ph
        sil = zeros > 1.0
        maskc = zeros
        tcfc = zeros
        for t in range(_MAX_GT):
            ox = jnp.minimum(pxr, sc[_R_GXR, t]) - jnp.maximum(pxl, sc[_R_GXL, t])
            oy = jnp.minimum(pyb, sc[_R_GYB, t]) - jnp.maximum(pyt, sc[_R_GYT, t])
            inter = jnp.maximum(ox, 0.0) * oy
            sil = sil | (inter > parea375 + sc[_R_G375, t])
            ii_m = jnp.where(sc[_R_BNV, t] == af, sc[_R_II, t], f32(-1.0))
            hit = (rowi == sc[_R_JJ, t]) & (coli == ii_m)
            maskc = jnp.where(hit, 1.0, maskc)
            tcfc = jnp.where(hit, sc[_R_TCONF, t], tcfc)
        conf = jax.nn.sigmoid(raw_blk[0, a, 4])
        scale = jnp.where(maskc > 0, _OBJECT_SCALE, jnp.where(sil, 0.0, 1.0))
        dconf = scale * (conf - tcfc)
        lconf = lconf + jnp.sum(dconf * dconf, axis=0, keepdims=True)

    pltpu.make_async_copy(wrow, swin, sem_w).wait()
    for t in range(_MAX_GT):
        pltpu.make_async_copy(gbuf.at[t], gbuf.at[t], sem_g).wait()

    lane64 = jax.lax.broadcasted_iota(jnp.int32, (1, _NW), 1).astype(f32)
    si80 = jax.lax.broadcasted_iota(jnp.int32, (_NUM_CLASSES, _NW), 0).astype(f32)
    lx = jnp.zeros((1, _NW), f32)
    ly = jnp.zeros((1, _NW), f32)
    lw = jnp.zeros((1, _NW), f32)
    lh = jnp.zeros((1, _NW), f32)
    lcls = jnp.zeros((1, _NW), f32)
    for t in range(_MAX_GT):
        wv = swin[0, t]
        ii_g = jnp.where(wv > 0, sc[_R_II, t], f32(-1.0))
        lm = lane64 == ii_g
        sxr = jax.nn.sigmoid(gbuf[t, 0:1, :])
        syr = jax.nn.sigmoid(gbuf[t, 1:2, :])
        rwr = gbuf[t, 2:3, :]
        rhr = gbuf[t, 3:4, :]
        lx = lx + jnp.where(lm, (sxr - sc[_R_TX, t]) ** 2, 0.0)
        ly = ly + jnp.where(lm, (syr - sc[_R_TY, t]) ** 2, 0.0)
        lw = lw + jnp.where(lm, (rwr - sc[_R_TW, t]) ** 2, 0.0)
        lh = lh + jnp.where(lm, (rhr - sc[_R_TH, t]) ** 2, 0.0)
        gcls = gbuf[t, 5:5 + _NUM_CLASSES, :]
        lse = jnp.log(jnp.sum(jnp.exp(gcls), axis=0, keepdims=True))
        pick = jnp.sum(jnp.where(si80 == sc[_R_TCLS, t], gcls, 0.0),
                       axis=0, keepdims=True)
        lcls = lcls + jnp.where(lm, (lse - pick) - _LOG80, 0.0)

    def pad128(r):
        return jnp.concatenate([r, jnp.zeros((1, _NL - _NW), f32)], axis=1)

    ccls = pad128(lcls) + f32(_NUM_ANCHORS * _NH * _NW * _LOG80 / _NL)

    stacked = jnp.concatenate(
        [pad128(lx), pad128(ly), pad128(lw), pad128(lh), lconf, ccls,
         validv, jnp.zeros((1, _NL), f32)], axis=0)
    part[0] = stacked


def kernel(output, target):
    nB = output.shape[0]
    out5 = output.reshape(nB, _NUM_ANCHORS, 5 + _NUM_CLASSES, _NH, _NW)
    raw = out5[:, :, :5].reshape(nB, _NUM_ANCHORS, 5, _NHS, _NL)
    tgt = target.reshape(nB, _MAX_GT, 5).transpose(0, 2, 1)
    tgt = jnp.pad(tgt, ((0, 0), (0, 0), (0, _TPAD - _MAX_GT)))
    tgt_flat = tgt.reshape(nB, 5 * _TPAD)

    part = pl.pallas_call(
        _region_loss_kernel,
        out_shape=jax.ShapeDtypeStruct((nB, 8, _NL), jnp.float32),
        grid=(nB,),
        in_specs=[
            pl.BlockSpec(memory_space=pltpu.SMEM),
            pl.BlockSpec((1, 5, _TPAD), lambda b: (b, 0, 0)),
            pl.BlockSpec((1, _NUM_ANCHORS, 5, _NHS, _NL), lambda b: (b, 0, 0, 0, 0)),
            pl.BlockSpec(memory_space=pl.ANY),
        ],
        out_specs=pl.BlockSpec((1, 8, _NL), lambda b: (b, 0, 0)),
        scratch_shapes=[
            pltpu.SMEM((_N_ROWS, _TPAD), jnp.float32),
            pltpu.SMEM((1, _TPAD), jnp.float32),
            pltpu.VMEM((1, _TPAD), jnp.float32),
            pltpu.VMEM((_MAX_GT, 5 + _NUM_CLASSES, _NW), jnp.float32),
            pltpu.SemaphoreType.DMA,
            pltpu.SemaphoreType.DMA,
        ],
        compiler_params=pltpu.CompilerParams(
            dimension_semantics=("parallel",),
        ),
        name="region_loss_sparse",
    )(tgt_flat, tgt, raw, out5)

    sums = jnp.sum(part, axis=(0, 2))
    ngt = sums[6]
    return (sums[0] + sums[1] + sums[2] + sums[3] + sums[4] + sums[5]) / ngt
